# Initial kernel scaffold; baseline (speedup 1.0000x reference)
import jax
import jax.numpy as jnp
from jax.experimental import pallas as pl


def kernel(points):
    raise NotImplementedError("write your pallas kernel here")


# speedup vs baseline: 9.2075x; 9.2075x over previous
import jax
import jax.numpy as jnp
from jax import lax
from jax.experimental import pallas as pl
from jax.experimental.pallas import tpu as pltpu
from jax.experimental.pallas import tpu_sc as plsc

N = 8
P = 16384
S = 2048
NC = 2
NS = 16
L = 16

ROWS_PER_CORE = N // NC
TILES_PER_ROW = NS // ROWS_PER_CORE
PTS_PER_TILE = P // TILES_PER_ROW
CHUNKS = PTS_PER_TILE // L

_GATHER_DNUMS = lax.GatherDimensionNumbers(
    offset_dims=(), collapsed_slice_dims=(0,), start_index_map=(0,))


def _take(vec, idx_vec):
    return lax.gather(vec, idx_vec[:, None], _GATHER_DNUMS, (1,),
                      mode=lax.GatherScatterMode.PROMISE_IN_BOUNDS)


def _bcast_lane(vec, lane):
    return _take(vec, jnp.full((L,), lane, jnp.int32))


def _fps_body(pts_hbm, idx_hbm, samp_hbm,
              x_ref, y_ref, z_ref, dist_ref, idx_buf, samp_x, samp_y, samp_z,
              recv_f, recv_i, pub_f, pub_i,
              shared_f, shared_i):
    c = lax.axis_index("c")
    s = lax.axis_index("s")
    rowslot = s // TILES_PER_ROW
    row = c * ROWS_PER_CORE + rowslot
    t = s % TILES_PER_ROW
    base = t * PTS_PER_TILE

    pltpu.sync_copy(pts_hbm.at[pl.ds(row * 3 * P, P)], x_ref)
    pltpu.sync_copy(pts_hbm.at[pl.ds(row * 3 * P + P, P)], y_ref)
    pltpu.sync_copy(pts_hbm.at[pl.ds(row * 3 * P + 2 * P, P)], z_ref)

    big = jnp.full((L,), 1e10, jnp.float32)

    def init_chunk(j, _):
        dist_ref[pl.ds(j * L, L)] = big
        return 0

    lax.fori_loop(0, CHUNKS, init_chunk, 0, unroll=8)

    zero16 = jnp.zeros((L,), jnp.int32)
    lane_iota = lax.broadcasted_iota(jnp.int32, (L,), 0)

    cx0 = _take(x_ref[pl.ds(0, L)], zero16)
    cy0 = _take(y_ref[pl.ds(0, L)], zero16)
    cz0 = _take(z_ref[pl.ds(0, L)], zero16)

    def fps_iter(i, carry):
        cx, cy, cz, idxv, acc = carry

        acc = jnp.where(lane_iota == (i & (L - 1)), idxv, acc)

        @pl.when(jnp.logical_and(t == 0, (i & (L - 1)) == (L - 1)))
        def _():
            idx_buf[pl.ds(i - (L - 1), L)] = acc

        def chunk(j, mk):
            m, k = mk
            sl = pl.ds(base + j * L, L)
            dx = x_ref[sl] - cx
            dy = y_ref[sl] - cy
            dz = z_ref[sl] - cz
            d = dx * dx + dy * dy + dz * dz
            dsl = pl.ds(j * L, L)
            dn = jnp.minimum(dist_ref[dsl], d)
            dist_ref[dsl] = dn
            upd = dn > m
            m = jnp.where(upd, dn, m)
            k = jnp.where(upd, jnp.full((L,), j, jnp.int32), k)
            return (m, k)

        m, k = lax.fori_loop(
            0, CHUNKS, chunk,
            (jnp.full((L,), -1.0, jnp.float32), zero16), unroll=4)

        mmax = jnp.max(m)
        cand = jnp.where(m == jnp.full((L,), mmax),
                         k * L + lane_iota, jnp.full((L,), P, jnp.int32))
        li = base + jnp.min(cand)
        liv = jnp.full((L,), li, jnp.int32)
        pxv = plsc.load_gather(x_ref, [liv])
        pyv = plsc.load_gather(y_ref, [liv])
        pzv = plsc.load_gather(z_ref, [liv])

        rec = jnp.full((L,), mmax, jnp.float32)
        rec = jnp.where(lane_iota == 1, pxv, rec)
        rec = jnp.where(lane_iota == 2, pyv, rec)
        rec = jnp.where(lane_iota == 3, pzv, rec)
        pub_f[:] = rec
        pub_i[:] = liv
        slot = (rowslot * TILES_PER_ROW + t) * L
        pltpu.sync_copy(pub_f, shared_f.at[pl.ds(slot, L)])
        pltpu.sync_copy(pub_i, shared_i.at[pl.ds(slot, L)])
        plsc.subcore_barrier()
        rbase = rowslot * TILES_PER_ROW * L
        pltpu.sync_copy(shared_f.at[pl.ds(rbase, TILES_PER_ROW * L)], recv_f)
        pltpu.sync_copy(shared_i.at[pl.ds(rbase, TILES_PER_ROW * L)], recv_i)
        plsc.subcore_barrier()

        best_f = recv_f[pl.ds(0, L)]
        best_i = recv_i[pl.ds(0, L)]
        best_m = _bcast_lane(best_f, 0)
        for tt in range(1, TILES_PER_ROW):
            rf = recv_f[pl.ds(tt * L, L)]
            ri = recv_i[pl.ds(tt * L, L)]
            rm = _bcast_lane(rf, 0)
            better = rm > best_m
            best_f = jnp.where(better, rf, best_f)
            best_i = jnp.where(better, ri, best_i)
            best_m = jnp.where(better, rm, best_m)

        ncx = _bcast_lane(best_f, 1)
        ncy = _bcast_lane(best_f, 2)
        ncz = _bcast_lane(best_f, 3)
        nidx = _bcast_lane(best_i, 0)
        return (ncx, ncy, ncz, nidx, acc)

    lax.fori_loop(0, S, fps_iter, (cx0, cy0, cz0, zero16, zero16))

    @pl.when(t == 0)
    def _():
        def out_chunk(j, _):
            sl = pl.ds(j * L, L)
            iv = idx_buf[sl]
            samp_x[sl] = plsc.load_gather(x_ref, [iv])
            samp_y[sl] = plsc.load_gather(y_ref, [iv])
            samp_z[sl] = plsc.load_gather(z_ref, [iv])
            return 0

        lax.fori_loop(0, S // L, out_chunk, 0, unroll=4)
        pltpu.sync_copy(idx_buf, idx_hbm.at[pl.ds(row * S, S)])
        pltpu.sync_copy(samp_x, samp_hbm.at[pl.ds(row * 3 * S, S)])
        pltpu.sync_copy(samp_y, samp_hbm.at[pl.ds(row * 3 * S + S, S)])
        pltpu.sync_copy(samp_z, samp_hbm.at[pl.ds(row * 3 * S + 2 * S, S)])


@jax.jit
def kernel(points):
    pts_t = points.transpose(0, 2, 1).reshape(N * 3 * P)

    fps = pl.kernel(
        _fps_body,
        out_type=(
            jax.ShapeDtypeStruct((N * S,), jnp.int32),
            jax.ShapeDtypeStruct((N * 3 * S,), jnp.float32),
        ),
        mesh=plsc.VectorSubcoreMesh(core_axis_name="c", subcore_axis_name="s",
                                    num_cores=NC, num_subcores=NS),
        compiler_params=pltpu.CompilerParams(needs_layout_passes=False),
        scratch_types=[
            pltpu.VMEM((P,), jnp.float32),
            pltpu.VMEM((P,), jnp.float32),
            pltpu.VMEM((P,), jnp.float32),
            pltpu.VMEM((PTS_PER_TILE,), jnp.float32),
            pltpu.VMEM((S,), jnp.int32),
            pltpu.VMEM((S,), jnp.float32),
            pltpu.VMEM((S,), jnp.float32),
            pltpu.VMEM((S,), jnp.float32),
            pltpu.VMEM((TILES_PER_ROW * L,), jnp.float32),
            pltpu.VMEM((TILES_PER_ROW * L,), jnp.int32),
            pltpu.VMEM((L,), jnp.float32),
            pltpu.VMEM((L,), jnp.int32),
            pltpu.VMEM_SHARED((ROWS_PER_CORE * TILES_PER_ROW * L,), jnp.float32),
            pltpu.VMEM_SHARED((ROWS_PER_CORE * TILES_PER_ROW * L,), jnp.int32),
        ],
    )
    idx_flat, samp_flat = fps(pts_t)
    indices = idx_flat.reshape(N, S)
    sampled = samp_flat.reshape(N, 3, S).transpose(0, 2, 1)
    return (indices, sampled)


# speedup vs baseline: 9.8477x; 1.0695x over previous
import jax
import jax.numpy as jnp
from jax import lax
from jax.experimental import pallas as pl
from jax.experimental.pallas import tpu as pltpu
from jax.experimental.pallas import tpu_sc as plsc

N = 8
P = 16384
S = 2048
NC = 2
NS = 16
L = 16

ROWS_PER_CORE = N // NC
TILES_PER_ROW = NS // ROWS_PER_CORE
PTS_PER_TILE = P // TILES_PER_ROW
CHUNKS = PTS_PER_TILE // L
UNROLL = 4
GROUPS = CHUNKS // UNROLL
SLOTS = ROWS_PER_CORE * TILES_PER_ROW * L

_GATHER_DNUMS = lax.GatherDimensionNumbers(
    offset_dims=(), collapsed_slice_dims=(0,), start_index_map=(0,))


def _take(vec, idx_vec):
    return lax.gather(vec, idx_vec[:, None], _GATHER_DNUMS, (1,),
                      mode=lax.GatherScatterMode.PROMISE_IN_BOUNDS)


def _bcast_lane(vec, lane):
    return _take(vec, jnp.full((L,), lane, jnp.int32))


def _fps_body(pts_hbm, idx_hbm, samp_hbm,
              x_ref, y_ref, z_ref, dist_ref, idx_buf, samp_x, samp_y, samp_z,
              recv_f, pub_f, shared_f):
    c = lax.axis_index("c")
    s = lax.axis_index("s")
    rowslot = s // TILES_PER_ROW
    row = c * ROWS_PER_CORE + rowslot
    t = s % TILES_PER_ROW
    base = t * PTS_PER_TILE

    pltpu.sync_copy(pts_hbm.at[pl.ds(row * 3 * P, P)], x_ref)
    pltpu.sync_copy(pts_hbm.at[pl.ds(row * 3 * P + P, P)], y_ref)
    pltpu.sync_copy(pts_hbm.at[pl.ds(row * 3 * P + 2 * P, P)], z_ref)

    big = jnp.full((L,), 1e10, jnp.float32)

    def init_chunk(j, _):
        dist_ref[pl.ds(j * L, L)] = big
        return 0

    lax.fori_loop(0, CHUNKS, init_chunk, 0, unroll=8)

    zero16 = jnp.zeros((L,), jnp.int32)
    lane_iota = lax.broadcasted_iota(jnp.int32, (L,), 0)

    cx0 = _take(x_ref[pl.ds(0, L)], zero16)
    cy0 = _take(y_ref[pl.ds(0, L)], zero16)
    cz0 = _take(z_ref[pl.ds(0, L)], zero16)

    def comb(a, b):
        va, ka = a
        vb, kb = b
        better = vb > va
        return (jnp.where(better, vb, va), jnp.where(better, kb, ka))

    def fps_iter(i, carry):
        cx, cy, cz, idxv, acc = carry

        acc = jnp.where(lane_iota == (i & (L - 1)), idxv, acc)

        @pl.when(jnp.logical_and(t == 0, (i & (L - 1)) == (L - 1)))
        def _():
            idx_buf[pl.ds(i - (L - 1), L)] = acc

        def group(g, mk):
            cands = []
            for u in range(UNROLL):
                j = g * UNROLL + u
                sl = pl.ds(base + j * L, L)
                dx = x_ref[sl] - cx
                dy = y_ref[sl] - cy
                dz = z_ref[sl] - cz
                d = dx * dx + dy * dy + dz * dz
                dsl = pl.ds(j * L, L)
                dn = jnp.minimum(dist_ref[dsl], d)
                dist_ref[dsl] = dn
                cands.append((dn, jnp.full((L,), j, jnp.int32)))
            c01 = comb(cands[0], cands[1])
            c23 = comb(cands[2], cands[3])
            return comb(mk, comb(c01, c23))

        m, k = lax.fori_loop(
            0, GROUPS, group,
            (jnp.full((L,), -1.0, jnp.float32), zero16), unroll=2)

        mmax = jnp.max(m)
        cand = jnp.where(m == jnp.full((L,), mmax),
                         k * L + lane_iota, jnp.full((L,), P, jnp.int32))
        li = base + jnp.min(cand)
        liv = jnp.full((L,), li, jnp.int32)
        pxv = plsc.load_gather(x_ref, [liv])
        pyv = plsc.load_gather(y_ref, [liv])
        pzv = plsc.load_gather(z_ref, [liv])

        rec = jnp.full((L,), mmax, jnp.float32)
        rec = jnp.where(lane_iota == 1, pxv, rec)
        rec = jnp.where(lane_iota == 2, pyv, rec)
        rec = jnp.where(lane_iota == 3, pzv, rec)
        rec = jnp.where(lane_iota == 4, plsc.bitcast(liv, jnp.float32), rec)
        pub_f[:] = rec
        buf = (i & 1) * SLOTS
        slot = buf + (rowslot * TILES_PER_ROW + t) * L
        pltpu.sync_copy(pub_f, shared_f.at[pl.ds(slot, L)])
        plsc.subcore_barrier()
        rbase = buf + rowslot * TILES_PER_ROW * L
        pltpu.sync_copy(shared_f.at[pl.ds(rbase, TILES_PER_ROW * L)], recv_f)

        best_f = recv_f[pl.ds(0, L)]
        best_m = _bcast_lane(best_f, 0)
        for tt in range(1, TILES_PER_ROW):
            rf = recv_f[pl.ds(tt * L, L)]
            rm = _bcast_lane(rf, 0)
            better = rm > best_m
            best_f = jnp.where(better, rf, best_f)
            best_m = jnp.where(better, rm, best_m)

        ncx = _bcast_lane(best_f, 1)
        ncy = _bcast_lane(best_f, 2)
        ncz = _bcast_lane(best_f, 3)
        nidx = plsc.bitcast(_bcast_lane(best_f, 4), jnp.int32)
        return (ncx, ncy, ncz, nidx, acc)

    lax.fori_loop(0, S, fps_iter, (cx0, cy0, cz0, zero16, zero16))

    @pl.when(t == 0)
    def _():
        def out_chunk(j, _):
            sl = pl.ds(j * L, L)
            iv = idx_buf[sl]
            samp_x[sl] = plsc.load_gather(x_ref, [iv])
            samp_y[sl] = plsc.load_gather(y_ref, [iv])
            samp_z[sl] = plsc.load_gather(z_ref, [iv])
            return 0

        lax.fori_loop(0, S // L, out_chunk, 0, unroll=4)
        pltpu.sync_copy(idx_buf, idx_hbm.at[pl.ds(row * S, S)])
        pltpu.sync_copy(samp_x, samp_hbm.at[pl.ds(row * 3 * S, S)])
        pltpu.sync_copy(samp_y, samp_hbm.at[pl.ds(row * 3 * S + S, S)])
        pltpu.sync_copy(samp_z, samp_hbm.at[pl.ds(row * 3 * S + 2 * S, S)])


@jax.jit
def kernel(points):
    pts_t = points.transpose(0, 2, 1).reshape(N * 3 * P)

    fps = pl.kernel(
        _fps_body,
        out_type=(
            jax.ShapeDtypeStruct((N * S,), jnp.int32),
            jax.ShapeDtypeStruct((N * 3 * S,), jnp.float32),
        ),
        mesh=plsc.VectorSubcoreMesh(core_axis_name="c", subcore_axis_name="s",
                                    num_cores=NC, num_subcores=NS),
        compiler_params=pltpu.CompilerParams(needs_layout_passes=False),
        scratch_types=[
            pltpu.VMEM((P,), jnp.float32),
            pltpu.VMEM((P,), jnp.float32),
            pltpu.VMEM((P,), jnp.float32),
            pltpu.VMEM((PTS_PER_TILE,), jnp.float32),
            pltpu.VMEM((S,), jnp.int32),
            pltpu.VMEM((S,), jnp.float32),
            pltpu.VMEM((S,), jnp.float32),
            pltpu.VMEM((S,), jnp.float32),
            pltpu.VMEM((TILES_PER_ROW * L,), jnp.float32),
            pltpu.VMEM((L,), jnp.float32),
            pltpu.VMEM_SHARED((2 * SLOTS,), jnp.float32),
        ],
    )
    idx_flat, samp_flat = fps(pts_t)
    indices = idx_flat.reshape(N, S)
    sampled = samp_flat.reshape(N, 3, S).transpose(0, 2, 1)
    return (indices, sampled)
